# baseline (speedup 1.0000x reference)
import functools

import jax
import jax.numpy as jnp
from jax import lax
from jax.experimental import pallas as pl
from jax.experimental.pallas import tpu as pltpu
from jax.experimental.pallas import tpu_sc as plsc

N = 10000
DEG = 32
D = 128

NC = 2
NS = 16
NW = NC * NS

CH = 4
RPC = CH * DEG
NB = 4

NPW0 = 640
NPW1 = 0
N_PAD = NS * (NPW0 + NPW1)
E_PAD = N_PAD * DEG


def _make_sc_gather_sum():
    mesh = plsc.VectorSubcoreMesh(core_axis_name="c", subcore_axis_name="s")
    nch_max = max(NPW0, NPW1) // CH

    @functools.partial(
        pl.kernel,
        mesh=mesh,
        out_type=jax.ShapeDtypeStruct((N_PAD, D), jnp.float32),
        scratch_types=[
            pltpu.VMEM((nch_max, RPC), jnp.int32),
            pltpu.VMEM((NB, RPC, D), jnp.float32),
            pltpu.VMEM((NB, CH, D), jnp.float32),
            pltpu.SemaphoreType.DMA,
            pltpu.SemaphoreType.DMA,
            pltpu.SemaphoreType.DMA,
            pltpu.SemaphoreType.DMA,
            pltpu.SemaphoreType.DMA,
            pltpu.SemaphoreType.DMA,
            pltpu.SemaphoreType.DMA,
            pltpu.SemaphoreType.DMA,
        ],
    )
    def sc_gather_sum(x_hbm, nbr_hbm, out_hbm, idx_v, buf, outb,
                      g0, g1, g2, g3, o0, o1, o2, o3):
        gsems = (g0, g1, g2, g3)
        osems = (o0, o1, o2, o3)
        cid = lax.axis_index("c")
        sid = lax.axis_index("s")

        def run(node_base, cbase, npw):
            nch = npw // CH

            pltpu.sync_copy(nbr_hbm.at[pl.ds(cbase, nch)],
                            idx_v.at[pl.ds(0, nch)])

            def gather(chunk, b):
                return pltpu.make_async_copy(
                    x_hbm.at[idx_v.at[chunk]], buf.at[b], gsems[b])

            def out_copy(chunk, b):
                return pltpu.make_async_copy(
                    outb.at[b],
                    out_hbm.at[pl.ds(node_base + chunk * CH, CH)],
                    osems[b])

            for b in range(NB):
                gather(b, b).start()

            def process(chunk, b):
                gather(chunk, b).wait()

                @pl.when(chunk >= NB)
                def _():
                    out_copy(chunk - NB, b).wait()

                for nd in range(CH):
                    base = nd * DEG

                    def row_body(r4, accs):
                        accs = list(accs)
                        for rr in range(4):
                            for k in range(8):
                                accs[k] = accs[k] + buf[
                                    b, base + r4 * 4 + rr, pl.ds(16 * k, 16)]
                        return tuple(accs)

                    accs = lax.fori_loop(
                        0, DEG // 4, row_body,
                        tuple(jnp.zeros((16,), jnp.float32) for _ in range(8)))
                    for k in range(8):
                        outb[b, nd, pl.ds(16 * k, 16)] = accs[k]
                out_copy(chunk, b).start()

                @pl.when(chunk + NB < nch)
                def _():
                    gather(chunk + NB, b).start()

            def group(p, carry):
                for b in range(NB):
                    process(p * NB + b, b)
                return carry

            lax.fori_loop(0, nch // NB, group, 0)

            for b in range(NB):
                out_copy(nch - NB + b, b).wait()

        if NPW0 > 0:
            @pl.when(cid == 0)
            def _():
                run(sid * NPW0, sid * (NPW0 // CH), NPW0)

        if NPW1 > 0:
            @pl.when(cid == 1)
            def _():
                run(NS * NPW0 + sid * NPW1,
                    NS * (NPW0 // CH) + sid * (NPW1 // CH), NPW1)

    return sc_gather_sum


@functools.cache
def _sc_gather_sum_cached():
    return _make_sc_gather_sum()

BR = 1000


def _tc_block(x_ref, s_ref, wg_ref, wl_ref, ws_ref, b_ref, o_ref):
    xb = x_ref[...]
    wc = wg_ref[...] + ws_ref[...]
    z = lax.dot_general(xb, wc, (((1,), (1,)), ((), ())),
                        preferred_element_type=jnp.float32)
    sb = s_ref[...] * (1.0 / DEG)
    z = z + lax.dot_general(sb, wl_ref[...], (((1,), (1,)), ((), ())),
                            preferred_element_type=jnp.float32)
    z = z + b_ref[...]
    o_ref[...] = jnp.where(z > 0.0, z, jnp.exp(jnp.minimum(z, 0.0)) - 1.0)


def _tc_fuse(x, s_pad, Wg, Wl, Ws, bias2d):
    return pl.pallas_call(
        _tc_block,
        grid=(N // BR,),
        in_specs=[
            pl.BlockSpec((BR, D), lambda i: (i, 0)),
            pl.BlockSpec((BR, D), lambda i: (i, 0)),
            pl.BlockSpec((D, D), lambda i: (0, 0)),
            pl.BlockSpec((D, D), lambda i: (0, 0)),
            pl.BlockSpec((D, D), lambda i: (0, 0)),
            pl.BlockSpec((1, D), lambda i: (0, 0)),
        ],
        out_specs=pl.BlockSpec((BR, D), lambda i: (i, 0)),
        out_shape=jax.ShapeDtypeStruct((N, D), jnp.float32),
    )(x, s_pad, Wg, Wl, Ws, bias2d)


def kernel(x, edge, neighbors, W_global, W_local, W_self, bias):
    pad = E_PAD - neighbors.shape[0]
    nbr = jnp.concatenate(
        [neighbors, jnp.zeros((pad,), jnp.int32)]).reshape(E_PAD // RPC, RPC)
    s_pad = _sc_gather_sum_cached()(x, nbr)
    return _tc_fuse(x, s_pad, W_global, W_local, W_self,
                    bias.reshape(1, D))


# speedup vs baseline: 1.4329x; 1.4329x over previous
import functools

import jax
import jax.numpy as jnp
import numpy as np
from jax import lax
from jax.experimental import pallas as pl
from jax.experimental.pallas import tpu as pltpu
from jax.experimental.pallas import tpu_sc as plsc

N = 10000
DEG = 32
D = 128

NC = 2
NS = 16
NW = NC * NS

CH = 4
RPC = CH * DEG
NB = 4

NPW0 = 320
NPW1 = 320
N_PAD = NS * (NPW0 + NPW1)
E_PAD = N_PAD * DEG


def _make_sc_gather_sum():
    mesh = plsc.VectorSubcoreMesh(core_axis_name="c", subcore_axis_name="s")
    nch_max = max(NPW0, NPW1) // CH

    @functools.partial(
        pl.kernel,
        mesh=mesh,
        compiler_params=pltpu.CompilerParams(use_tc_tiling_on_sc=False),
        out_type=jax.ShapeDtypeStruct((N_PAD, D), jnp.float32),
        scratch_types=[
            pltpu.VMEM((nch_max, RPC), jnp.int32),
            pltpu.VMEM((NB, RPC, D // 2), jnp.int32),
            pltpu.VMEM((NB, CH, D), jnp.float32),
            pltpu.SemaphoreType.DMA,
            pltpu.SemaphoreType.DMA,
            pltpu.SemaphoreType.DMA,
            pltpu.SemaphoreType.DMA,
            pltpu.SemaphoreType.DMA,
            pltpu.SemaphoreType.DMA,
            pltpu.SemaphoreType.DMA,
            pltpu.SemaphoreType.DMA,
        ],
    )
    def sc_gather_sum(x_hbm, nbr_hbm, out_hbm, idx_v, buf, outb,
                      g0, g1, g2, g3, o0, o1, o2, o3):
        gsems = (g0, g1, g2, g3)
        osems = (o0, o1, o2, o3)
        cid = lax.axis_index("c")
        sid = lax.axis_index("s")

        def run(node_base, cbase, npw):
            nch = npw // CH

            pltpu.sync_copy(nbr_hbm.at[pl.ds(cbase, nch)],
                            idx_v.at[pl.ds(0, nch)])

            def gather(chunk, b):
                return pltpu.make_async_copy(
                    x_hbm.at[idx_v.at[chunk]], buf.at[b], gsems[b])

            def out_copy(chunk, b):
                return pltpu.make_async_copy(
                    outb.at[b],
                    out_hbm.at[pl.ds(node_base + chunk * CH, CH)],
                    osems[b])

            for b in range(NB):
                gather(b, b).start()

            def process(chunk, b):
                gather(chunk, b).wait()

                @pl.when(chunk >= NB)
                def _():
                    out_copy(chunk - NB, b).wait()

                for nd in range(CH):
                    base = nd * DEG

                    def row_body(r2, accs):
                        accs = list(accs)
                        for rr in range(2):
                            r = base + 2 * r2 + rr
                            for k4 in range(4):
                                w = buf[b, r, pl.ds(16 * k4, 16)]
                                lo = lax.bitcast_convert_type(
                                    w << 16, jnp.float32)
                                hi = lax.bitcast_convert_type(
                                    w & jnp.int32(-65536), jnp.float32)
                                accs[2 * k4] = accs[2 * k4] + lo
                                accs[2 * k4 + 1] = accs[2 * k4 + 1] + hi
                        return tuple(accs)

                    accs = lax.fori_loop(
                        0, DEG // 2, row_body,
                        tuple(jnp.zeros((16,), jnp.float32) for _ in range(8)))
                    for k in range(8):
                        outb[b, nd, pl.ds(16 * k, 16)] = accs[k]
                out_copy(chunk, b).start()

                @pl.when(chunk + NB < nch)
                def _():
                    gather(chunk + NB, b).start()

            def group(p, carry):
                for b in range(NB):
                    process(p * NB + b, b)
                return carry

            lax.fori_loop(0, nch // NB, group, 0)

            for b in range(NB):
                out_copy(nch - NB + b, b).wait()

        if NPW0 > 0:
            @pl.when(cid == 0)
            def _():
                run(sid * NPW0, sid * (NPW0 // CH), NPW0)

        if NPW1 > 0:
            @pl.when(cid == 1)
            def _():
                run(NS * NPW0 + sid * NPW1,
                    NS * (NPW0 // CH) + sid * (NPW1 // CH), NPW1)

    return sc_gather_sum


@functools.cache
def _sc_gather_sum_cached():
    return _make_sc_gather_sum()

BR = 1000

_COLS = np.arange(D)
_PERM = 32 * (_COLS // 32) + 2 * (_COLS % 16) + (_COLS % 32) // 16


def _tc_block(x_ref, s_ref, wg_ref, wl_ref, ws_ref, b_ref, o_ref):
    xb = x_ref[...]
    wc = wg_ref[...] + ws_ref[...]
    z = lax.dot_general(xb, wc, (((1,), (1,)), ((), ())),
                        preferred_element_type=jnp.float32)
    z = z + lax.dot_general(s_ref[...], wl_ref[...], (((1,), (1,)), ((), ())),
                            preferred_element_type=jnp.float32)
    z = z + b_ref[...]
    o_ref[...] = jnp.where(z > 0.0, z, jnp.exp(jnp.minimum(z, 0.0)) - 1.0)


def _tc_fuse(x, s_pad, Wg, Wl, Ws, bias2d):
    return pl.pallas_call(
        _tc_block,
        grid=(N // BR,),
        in_specs=[
            pl.BlockSpec((BR, D), lambda i: (i, 0)),
            pl.BlockSpec((BR, D), lambda i: (i, 0)),
            pl.BlockSpec((D, D), lambda i: (0, 0)),
            pl.BlockSpec((D, D), lambda i: (0, 0)),
            pl.BlockSpec((D, D), lambda i: (0, 0)),
            pl.BlockSpec((1, D), lambda i: (0, 0)),
        ],
        out_specs=pl.BlockSpec((BR, D), lambda i: (i, 0)),
        out_shape=jax.ShapeDtypeStruct((N, D), jnp.float32),
    )(x, s_pad, Wg, Wl, Ws, bias2d)


def kernel(x, edge, neighbors, W_global, W_local, W_self, bias):
    pad = E_PAD - neighbors.shape[0]
    nbr = jnp.concatenate(
        [neighbors, jnp.zeros((pad,), jnp.int32)]).reshape(E_PAD // RPC, RPC)
    x16 = x.astype(jnp.bfloat16)
    xi = jax.lax.bitcast_convert_type(
        x16.reshape(N, D // 2, 2), jnp.int32)
    s_perm = _sc_gather_sum_cached()(xi, nbr)
    wl_perm = W_local[:, _PERM] * (1.0 / DEG)
    return _tc_fuse(x, s_perm, W_global, wl_perm, W_self,
                    bias.reshape(1, D))
